# baseline (speedup 1.0000x reference)
import functools

import jax
import jax.numpy as jnp
from jax import lax
from jax.experimental import pallas as pl
from jax.experimental.pallas import tpu as pltpu
from jax.experimental.pallas import tpu_sc as plsc

E, CIN, COUT, D = 8192, 4, 8, 512
L = 16
NC, NS = 2, 16
NW = NC * NS
EPW = E // NW
C = 4
NCH = EPW // C
NB = 4
NBO = 2
JPE = D // L
INV_SQRT3 = 0.5773502691896258


def _body(x_hbm, y_hbm, o_hbm, xv, yv, ov,
          sx0, sx1, sx2, sx3, sy0, sy1, sy2, sy3, so0, so1):
    sx = (sx0, sx1, sx2, sx3)
    sy = (sy0, sy1, sy2, sy3)
    so = (so0, so1)
    wid = lax.axis_index("s") * NC + lax.axis_index("c")
    base = wid * EPW

    for b in range(NB):
        off = base + b * C
        pltpu.async_copy(x_hbm.at[pl.ds(off, C)], xv.at[b], sx[b])
        pltpu.async_copy(y_hbm.at[pl.ds(off, C)], yv.at[b], sy[b])

    def round_body(g, carry):
        for b in range(NB):
            bo = b % NBO
            ci = g * NB + b
            off = base + ci * C

            pltpu.make_async_copy(x_hbm.at[pl.ds(off, C)], xv.at[b], sx[b]).wait()
            pltpu.make_async_copy(y_hbm.at[pl.ds(off, C)], yv.at[b], sy[b]).wait()

            def _wait_out():
                pltpu.make_async_copy(
                    ov.at[bo], o_hbm.at[pl.ds(base, C)], so[bo]).wait()
            if b < NBO:
                pl.when(g > 0)(_wait_out)
            else:
                _wait_out()

            def _edge(e, carry3):
                for j in range(JPE):
                    s = pl.ds(j * L, L)
                    x0 = xv[b, e, 0, s]
                    x1 = xv[b, e, 1, s]
                    x2 = xv[b, e, 2, s]
                    x3 = xv[b, e, 3, s]
                    y0 = yv[b, e, 0, s]
                    y1 = yv[b, e, 1, s]
                    y2 = yv[b, e, 2, s]
                    y3 = yv[b, e, 3, s]
                    ov[bo, e, 0, s] = x0 * y0
                    ov[bo, e, 1, s] = x0 * y1
                    ov[bo, e, 2, s] = x0 * y2
                    ov[bo, e, 3, s] = x0 * y3
                    ov[bo, e, 4, s] = x1 * y0
                    ov[bo, e, 5, s] = x2 * y0
                    ov[bo, e, 6, s] = x3 * y0
                    ov[bo, e, 7, s] = (x1 * y1 + x2 * y2 + x3 * y3) * INV_SQRT3
                return carry3

            lax.fori_loop(0, C, _edge, 0)

            pltpu.async_copy(ov.at[bo], o_hbm.at[pl.ds(off, C)], so[bo])

            @pl.when(ci + NB < NCH)
            def _():
                noff = off + NB * C
                pltpu.async_copy(x_hbm.at[pl.ds(noff, C)], xv.at[b], sx[b])
                pltpu.async_copy(y_hbm.at[pl.ds(noff, C)], yv.at[b], sy[b])

        return carry

    lax.fori_loop(0, NCH // NB, round_body, 0)

    for bo in range(NBO):
        pltpu.make_async_copy(ov.at[bo], o_hbm.at[pl.ds(base, C)], so[bo]).wait()


_tp = functools.partial(
    pl.kernel,
    mesh=plsc.VectorSubcoreMesh(core_axis_name="c", subcore_axis_name="s"),
    out_type=jax.ShapeDtypeStruct((E, COUT, D), jnp.float32),
    scratch_types=[
        pltpu.VMEM((NB, C, CIN, D), jnp.float32),
        pltpu.VMEM((NB, C, CIN, D), jnp.float32),
        pltpu.VMEM((NBO, C, COUT, D), jnp.float32),
        pltpu.SemaphoreType.DMA,
        pltpu.SemaphoreType.DMA,
        pltpu.SemaphoreType.DMA,
        pltpu.SemaphoreType.DMA,
        pltpu.SemaphoreType.DMA,
        pltpu.SemaphoreType.DMA,
        pltpu.SemaphoreType.DMA,
        pltpu.SemaphoreType.DMA,
        pltpu.SemaphoreType.DMA,
        pltpu.SemaphoreType.DMA,
    ],
)(_body)


def kernel(x, y):
    return _tp(x, y)


# speedup vs baseline: 1.1013x; 1.1013x over previous
import functools

import jax
import jax.numpy as jnp
from jax import lax
from jax.experimental import pallas as pl
from jax.experimental.pallas import tpu as pltpu
from jax.experimental.pallas import tpu_sc as plsc

E, CIN, COUT, D = 8192, 4, 8, 512
L = 16
NC, NS = 2, 16
NW = NC * NS
EPW = E // NW
C = 4
NCH = EPW // C
NB = 4
JPE = D // L
INV_SQRT3 = 0.5773502691896258


def _body(x_hbm, y_hbm, o_hbm, xv, yv,
          sx0, sx1, sx2, sx3, sy0, sy1, sy2, sy3,
          sox0, sox1, sox2, sox3, soy0, soy1, soy2, soy3):
    sx = (sx0, sx1, sx2, sx3)
    sy = (sy0, sy1, sy2, sy3)
    sox = (sox0, sox1, sox2, sox3)
    soy = (soy0, soy1, soy2, soy3)
    wid = lax.axis_index("s") * NC + lax.axis_index("c")
    base = wid * EPW

    for b in range(NB):
        off = base + b * C
        pltpu.async_copy(x_hbm.at[pl.ds(off, C)], xv.at[b], sx[b])
        pltpu.async_copy(y_hbm.at[pl.ds(off, C)], yv.at[b], sy[b])

    def round_body(g, carry):
        for b in range(NB):
            ci = g * NB + b
            off = base + ci * C

            pltpu.make_async_copy(x_hbm.at[pl.ds(off, C)], xv.at[b], sx[b]).wait()
            pltpu.make_async_copy(y_hbm.at[pl.ds(off, C)], yv.at[b], sy[b]).wait()

            def _edge(e, carry3):
                for j in range(JPE):
                    s = pl.ds(j * L, L)
                    x0 = xv[b, e, 0, s]
                    x1 = xv[b, e, 1, s]
                    x2 = xv[b, e, 2, s]
                    x3 = xv[b, e, 3, s]
                    y0 = yv[b, e, 0, s]
                    y1 = yv[b, e, 1, s]
                    y2 = yv[b, e, 2, s]
                    y3 = yv[b, e, 3, s]
                    xv[b, e, 0, s] = x0 * y0
                    xv[b, e, 1, s] = x0 * y1
                    xv[b, e, 2, s] = x0 * y2
                    xv[b, e, 3, s] = x0 * y3
                    yv[b, e, 0, s] = x1 * y0
                    yv[b, e, 1, s] = x2 * y0
                    yv[b, e, 2, s] = x3 * y0
                    yv[b, e, 3, s] = (x1 * y1 + x2 * y2 + x3 * y3) * INV_SQRT3
                return carry3

            lax.fori_loop(0, C, _edge, 0)

            pltpu.async_copy(xv.at[b], o_hbm.at[pl.ds(off, C), pl.ds(0, CIN)], sox[b])
            pltpu.async_copy(yv.at[b], o_hbm.at[pl.ds(off, C), pl.ds(CIN, CIN)], soy[b])

            pb = (b - 2) % NB
            pci = ci - 2

            def _drain_refill():
                poff = base + pci * C
                pltpu.make_async_copy(
                    xv.at[pb], o_hbm.at[pl.ds(poff, C), pl.ds(0, CIN)], sox[pb]).wait()
                pltpu.make_async_copy(
                    yv.at[pb], o_hbm.at[pl.ds(poff, C), pl.ds(CIN, CIN)], soy[pb]).wait()

                @pl.when(pci + NB < NCH)
                def _():
                    noff = poff + NB * C
                    pltpu.async_copy(x_hbm.at[pl.ds(noff, C)], xv.at[pb], sx[pb])
                    pltpu.async_copy(y_hbm.at[pl.ds(noff, C)], yv.at[pb], sy[pb])

            if b < 2:
                pl.when(g > 0)(_drain_refill)
            else:
                _drain_refill()

        return carry

    lax.fori_loop(0, NCH // NB, round_body, 0)

    for ci in (NCH - 2, NCH - 1):
        b = ci % NB
        off = base + ci * C
        pltpu.make_async_copy(
            xv.at[b], o_hbm.at[pl.ds(off, C), pl.ds(0, CIN)], sox[b]).wait()
        pltpu.make_async_copy(
            yv.at[b], o_hbm.at[pl.ds(off, C), pl.ds(CIN, CIN)], soy[b]).wait()


_tp = functools.partial(
    pl.kernel,
    mesh=plsc.VectorSubcoreMesh(core_axis_name="c", subcore_axis_name="s"),
    out_type=jax.ShapeDtypeStruct((E, COUT, D), jnp.float32),
    scratch_types=[
        pltpu.VMEM((NB, C, CIN, D), jnp.float32),
        pltpu.VMEM((NB, C, CIN, D), jnp.float32),
        pltpu.SemaphoreType.DMA,
        pltpu.SemaphoreType.DMA,
        pltpu.SemaphoreType.DMA,
        pltpu.SemaphoreType.DMA,
        pltpu.SemaphoreType.DMA,
        pltpu.SemaphoreType.DMA,
        pltpu.SemaphoreType.DMA,
        pltpu.SemaphoreType.DMA,
        pltpu.SemaphoreType.DMA,
        pltpu.SemaphoreType.DMA,
        pltpu.SemaphoreType.DMA,
        pltpu.SemaphoreType.DMA,
        pltpu.SemaphoreType.DMA,
        pltpu.SemaphoreType.DMA,
        pltpu.SemaphoreType.DMA,
        pltpu.SemaphoreType.DMA,
    ],
)(_body)


def kernel(x, y):
    return _tp(x, y)
